# baseline (speedup 1.0000x reference)
import functools

import jax
import jax.numpy as jnp
from jax import lax
from jax.experimental import pallas as pl
from jax.experimental.pallas import tpu as pltpu
from jax.experimental.pallas import tpu_sc as plsc

NC = 2
NS = 16
L = 16
NW = NC * NS

B = 1024
CHUNK = 128
KSUB = B // CHUNK


@functools.partial(jax.jit, static_argnames=("h", "w"))
def _build_table(zt5, *, h, w):
    yb_n = h // 8
    xb_n = w // 128
    units = yb_n * xb_n
    per_w = units // NW
    mesh = plsc.VectorSubcoreMesh(core_axis_name="c", subcore_axis_name="s")

    @functools.partial(
        pl.kernel,
        out_type=jax.ShapeDtypeStruct((h, xb_n, 1024), jnp.float32),
        mesh=mesh,
        scratch_types=[
            pltpu.VMEM((2, 8, 1033), jnp.float32),
            pltpu.VMEM((2, 8, 1024), jnp.float32),
            pltpu.SemaphoreType.DMA,
            pltpu.SemaphoreType.DMA,
            pltpu.SemaphoreType.DMA,
            pltpu.SemaphoreType.DMA,
        ],
        compiler_params=pltpu.CompilerParams(
            needs_layout_passes=False, use_tc_tiling_on_sc=False),
    )
    def k(zt_hbm, tab_hbm, in_buf, out_buf, sem_i0, sem_i1, sem_o0, sem_o1):
        wid = lax.axis_index("s") * NC + lax.axis_index("c")
        ubase = wid * per_w
        iota = lax.iota(jnp.int32, L)
        cvec = iota % 8
        xpair = iota // 8
        sem_i = [sem_i0, sem_i1]
        sem_o = [sem_o0, sem_o1]

        def unit_yx(u):
            uu = ubase + u
            return uu // xb_n, uu % xb_n

        def make_in(u, p):
            yb, xb = unit_yx(u)
            return pltpu.make_async_copy(
                zt_hbm.at[:, yb, xb], in_buf.at[p, :, pl.ds(0, 1024)], sem_i[p])

        def make_out(u, p):
            yb, xb = unit_yx(u)
            return pltpu.make_async_copy(
                out_buf.at[p], tab_hbm.at[pl.ds(yb * 8, 8), xb], sem_o[p])

        def compute(u, p):
            @pl.loop(0, 8)
            def _yr(yr):
                pos0 = yr * 128 + xpair
                for xp in range(64):
                    g = plsc.load_gather(
                        in_buf.at[p], [cvec, pos0 + 2 * xp])
                    out_buf[p, yr, pl.ds(xp * 2 * 8, L)] = g

        make_in(0, 0).start()
        make_in(1, 1).start()

        @pl.loop(0, per_w, step=2)
        def _u(u):
            for par in range(2):
                cur = u + par
                make_in(cur, par).wait()

                @pl.when(cur >= 2)
                def _():
                    make_out(cur - 2, par).wait()

                compute(cur, par)

                @pl.when(cur + 2 < per_w)
                def _():
                    make_in(cur + 2, par).start()

                make_out(cur, par).start()

        make_out(per_w - 2, 0).wait()
        make_out(per_w - 1, 1).wait()

    return k(zt5)


@functools.partial(jax.jit, static_argnames=("n", "c"))
def _interp(zrs, idx128, w128, *, n, c):
    per_w = n // NW
    nblk = per_w // B
    rows_blk = B * 4 // CHUNK
    mesh = plsc.VectorSubcoreMesh(core_axis_name="c", subcore_axis_name="s")

    @functools.partial(
        pl.kernel,
        out_type=jax.ShapeDtypeStruct((n // CHUNK, c, CHUNK), jnp.float32),
        mesh=mesh,
        scratch_types=[
            pltpu.VMEM((rows_blk, CHUNK), jnp.int32),
            pltpu.VMEM((rows_blk, CHUNK), jnp.float32),
            pltpu.VMEM((B * 4, 8), jnp.float32),
            pltpu.VMEM((B, 17), jnp.float32),
            pltpu.VMEM((KSUB, 8, CHUNK), jnp.float32),
            pltpu.SemaphoreType.DMA,
        ],
        compiler_params=pltpu.CompilerParams(
            needs_layout_passes=False, use_tc_tiling_on_sc=False),
    )
    def k(zrs_hbm, idx_hbm, w_hbm, out_hbm,
          idx_buf, w_buf, g_buf, t_buf, o_buf, sem_g):
        wid = lax.axis_index("s") * NC + lax.axis_index("c")
        base = wid * per_w
        iota = lax.iota(jnp.int32, L)
        rsel = jnp.where(iota < 8, 0, 1).astype(jnp.int32)
        cmir = jnp.where(iota < 8, iota, 15 - iota).astype(jnp.int32)
        csplat = [jnp.full((L,), cc, jnp.int32) for cc in range(8)]

        @pl.loop(0, nblk)
        def _blk(s):
            qoff = pl.multiple_of(base + s * B, B)
            roff = pl.multiple_of(qoff // 32, rows_blk)
            coff = pl.multiple_of(qoff // CHUNK, KSUB)
            pltpu.sync_copy(idx_hbm.at[pl.ds(roff, rows_blk)], idx_buf)
            pltpu.sync_copy(w_hbm.at[pl.ds(roff, rows_blk)], w_buf)

            @pl.loop(0, rows_blk)
            def _fire(r):
                pltpu.async_copy(
                    zrs_hbm.at[idx_buf.at[r]],
                    g_buf.at[pl.ds(r * CHUNK, CHUNK), :], sem_g)

            @pl.loop(0, rows_blk)
            def _drain(r):
                pltpu.make_async_copy(
                    zrs_hbm.at[idx_buf.at[r]],
                    g_buf.at[pl.ds(r * CHUNK, CHUNK), :], sem_g).wait()

            @pl.loop(0, B, unroll=4)
            def _acc(q):
                q4 = q * 4
                qrow = q // 32
                col0 = q4 - qrow * CHUNK
                wrow = w_buf.at[qrow]
                v0 = plsc.load_gather(g_buf, [q4 + rsel, cmir])
                v1 = plsc.load_gather(g_buf, [q4 + 2 + rsel, cmir])
                w01 = plsc.load_gather(wrow, [col0 + rsel])
                w23 = plsc.load_gather(wrow, [col0 + 2 + rsel])
                s0 = v0 * w01 + v1 * w23
                t_buf[q, pl.ds(0, L)] = s0 + lax.rev(s0, (0,))

            @pl.loop(0, B // L, unroll=2)
            def _tr(u):
                kk = u // 8
                t = u - kk * 8
                qv = u * L + iota
                for cc in range(8):
                    o_buf[kk, cc, pl.ds(t * L, L)] = plsc.load_gather(
                        t_buf, [qv, csplat[cc]])

            pltpu.sync_copy(o_buf, out_hbm.at[pl.ds(coff, KSUB)])

    out3 = k(zrs, idx128, w128)
    return out3.transpose(1, 0, 2).reshape(c, n)


def kernel(z, weights, index):
    c, hh, ww = z.shape
    n = index.shape[0]
    zt5 = z.reshape(c, hh // 8, 8, ww // 128, 128).transpose(
        0, 1, 3, 2, 4).reshape(c, hh // 8, ww // 128, 1024)
    tab = _build_table(zt5, h=hh, w=ww)
    zrs = tab.reshape(hh * ww, c)
    idx128 = index.reshape(n * 4 // CHUNK, CHUNK)
    w128 = weights.reshape(n * 4 // CHUNK, CHUNK)
    return _interp(zrs, idx128, w128, n=n, c=c)


# speedup vs baseline: 3.5114x; 3.5114x over previous
import functools

import jax
import jax.numpy as jnp
from jax import lax
from jax.experimental import pallas as pl
from jax.experimental.pallas import tpu as pltpu
from jax.experimental.pallas import tpu_sc as plsc

NC = 2
NS = 16
L = 16
NW = NC * NS

B = 1024
CHUNK = 128
KSUB = B // CHUNK


@functools.partial(jax.jit, static_argnames=("h", "w"))
def _build_table(zt5, *, h, w):
    yb_n = h // 8
    xb_n = w // 128
    units = yb_n * xb_n
    per_w = units // NW
    mesh = plsc.VectorSubcoreMesh(core_axis_name="c", subcore_axis_name="s")

    @functools.partial(
        pl.kernel,
        out_type=jax.ShapeDtypeStruct((h, xb_n, 1024), jnp.float32),
        mesh=mesh,
        scratch_types=[
            pltpu.VMEM((2, 8, 1033), jnp.float32),
            pltpu.VMEM((2, 8, 1024), jnp.float32),
            pltpu.SemaphoreType.DMA,
            pltpu.SemaphoreType.DMA,
            pltpu.SemaphoreType.DMA,
            pltpu.SemaphoreType.DMA,
        ],
        compiler_params=pltpu.CompilerParams(
            needs_layout_passes=False, use_tc_tiling_on_sc=False),
    )
    def k(zt_hbm, tab_hbm, in_buf, out_buf, sem_i0, sem_i1, sem_o0, sem_o1):
        wid = lax.axis_index("s") * NC + lax.axis_index("c")
        ubase = wid * per_w
        iota = lax.iota(jnp.int32, L)
        cvec = iota % 8
        xpair = iota // 8
        sem_i = [sem_i0, sem_i1]
        sem_o = [sem_o0, sem_o1]

        def unit_yx(u):
            uu = ubase + u
            return uu // xb_n, uu % xb_n

        def make_in(u, p):
            yb, xb = unit_yx(u)
            return pltpu.make_async_copy(
                zt_hbm.at[:, yb, xb], in_buf.at[p, :, pl.ds(0, 1024)], sem_i[p])

        def make_out(u, p):
            yb, xb = unit_yx(u)
            return pltpu.make_async_copy(
                out_buf.at[p], tab_hbm.at[pl.ds(yb * 8, 8), xb], sem_o[p])

        def compute(u, p):
            @pl.loop(0, 8)
            def _yr(yr):
                pos0 = yr * 128 + xpair
                for xp in range(64):
                    g = plsc.load_gather(
                        in_buf.at[p], [cvec, pos0 + 2 * xp])
                    out_buf[p, yr, pl.ds(xp * 2 * 8, L)] = g

        make_in(0, 0).start()
        make_in(1, 1).start()

        @pl.loop(0, per_w, step=2)
        def _u(u):
            for par in range(2):
                cur = u + par
                make_in(cur, par).wait()

                @pl.when(cur >= 2)
                def _():
                    make_out(cur - 2, par).wait()

                compute(cur, par)

                @pl.when(cur + 2 < per_w)
                def _():
                    make_in(cur + 2, par).start()

                make_out(cur, par).start()

        make_out(per_w - 2, 0).wait()
        make_out(per_w - 1, 1).wait()

    return k(zt5)


@functools.partial(jax.jit, static_argnames=("n", "c"))
def _interp(zrs, idx128, w128, *, n, c):
    per_w = n // NW
    nblk = per_w // B
    rows_blk = B * 4 // CHUNK
    mesh = plsc.VectorSubcoreMesh(core_axis_name="c", subcore_axis_name="s")

    @functools.partial(
        pl.kernel,
        out_type=jax.ShapeDtypeStruct((n // CHUNK, c, CHUNK), jnp.float32),
        mesh=mesh,
        scratch_types=[
            pltpu.VMEM((4, 1028), jnp.int32),
            pltpu.VMEM((rows_blk, CHUNK), jnp.int32),
            pltpu.VMEM((4, B), jnp.float32),
            pltpu.VMEM((B * 4, 8), jnp.float32),
            pltpu.VMEM((B, 17), jnp.float32),
            pltpu.VMEM((KSUB, 8, CHUNK), jnp.float32),
            pltpu.SemaphoreType.DMA,
        ],
        compiler_params=pltpu.CompilerParams(
            needs_layout_passes=False, use_tc_tiling_on_sc=False),
    )
    def k(zrs_hbm, idx_hbm, w_hbm, out_hbm,
          idx_buf_t, idx_il, w_buf, g_buf, t_buf, o_buf, sem_g):
        wid = lax.axis_index("s") * NC + lax.axis_index("c")
        base = wid * per_w
        iota = lax.iota(jnp.int32, L)
        rsel = jnp.where(iota < 8, 0, 1).astype(jnp.int32)
        cmir = jnp.where(iota < 8, iota, 15 - iota).astype(jnp.int32)
        islo = iota < 8
        jsel = iota % 4
        qsel = iota // 4
        csplat = [jnp.full((L,), cc, jnp.int32) for cc in range(8)]

        @pl.loop(0, nblk)
        def _blk(s):
            qoff = pl.multiple_of(base + s * B, B)
            coff = pl.multiple_of(qoff // CHUNK, KSUB)
            pltpu.sync_copy(idx_hbm.at[:, pl.ds(qoff, B)],
                            idx_buf_t.at[:, pl.ds(0, B)])
            pltpu.sync_copy(w_hbm.at[:, pl.ds(qoff, B)], w_buf)

            @pl.loop(0, B * 4 // L, unroll=4)
            def _il(f):
                f0 = f * L
                v = plsc.load_gather(idx_buf_t, [jsel, (f0 // 4) + qsel])
                idx_il[f0 // CHUNK, pl.ds(f0 % CHUNK, L)] = v

            @pl.loop(0, rows_blk)
            def _fire(r):
                pltpu.async_copy(
                    zrs_hbm.at[idx_il.at[r]],
                    g_buf.at[pl.ds(r * CHUNK, CHUNK), :], sem_g)

            @pl.loop(0, rows_blk)
            def _drain(r):
                pltpu.make_async_copy(
                    zrs_hbm.at[idx_il.at[r]],
                    g_buf.at[pl.ds(r * CHUNK, CHUNK), :], sem_g).wait()

            @pl.loop(0, B // L)
            def _accg(qg):
                q0 = qg * L
                wv = [w_buf[j, pl.ds(q0, L)] for j in range(4)]
                ylv = wv[0] + wv[1]
                yuv = wv[2] + wv[3]
                xlv = wv[0] + wv[2]
                xuv = wv[1] + wv[3]
                for t in range(L):
                    q = q0 + t
                    q4 = q * 4
                    wx = jnp.where(islo, xlv[t], xuv[t])
                    v0 = plsc.load_gather(g_buf, [q4 + rsel, cmir])
                    v1 = plsc.load_gather(g_buf, [q4 + 2 + rsel, cmir])
                    s0 = (ylv[t] * v0 + yuv[t] * v1) * wx
                    t_buf[q, pl.ds(0, L)] = s0 + lax.rev(s0, (0,))

            @pl.loop(0, B // L, unroll=2)
            def _tr(u):
                kk = u // 8
                t = u - kk * 8
                qv = u * L + iota
                for cc in range(8):
                    o_buf[kk, cc, pl.ds(t * L, L)] = plsc.load_gather(
                        t_buf, [qv, csplat[cc]])

            pltpu.sync_copy(o_buf, out_hbm.at[pl.ds(coff, KSUB)])

    out3 = k(zrs, idx128, w128)
    return out3.transpose(1, 0, 2).reshape(c, n)


def kernel(z, weights, index):
    c, hh, ww = z.shape
    n = index.shape[0]
    zt5 = z.reshape(c, hh // 8, 8, ww // 128, 128).transpose(
        0, 1, 3, 2, 4).reshape(c, hh // 8, ww // 128, 1024)
    tab = _build_table(zt5, h=hh, w=ww)
    zrs = tab.reshape(hh * ww, c)
    return _interp(zrs, index.T, weights.T, n=n, c=c)
